# baseline (speedup 1.0000x reference)
import functools
import math

import jax
import jax.numpy as jnp
from jax.experimental import pallas as pl
from jax.experimental.pallas import tpu as pltpu

_TEMP = math.e
_LANES = 128


def _routing_body(x_ref, gw_ref, gb_ref, coef_ref, *, n_active, n_camp_a):
    scores = jnp.dot(x_ref[...], gw_ref[...], preferred_element_type=jnp.float32)
    scores = (scores + gb_ref[...]) * (1.0 / _TEMP)
    m = jnp.max(scores, axis=-1, keepdims=True)
    ex = jnp.exp(scores - m)
    probs = ex / jnp.sum(ex, axis=-1, keepdims=True)

    lane = jax.lax.broadcasted_iota(jnp.int32, scores.shape, 1)
    rank = jnp.zeros(scores.shape, jnp.float32)
    for j in range(8):
        sj = scores[:, j : j + 1]
        gt = (sj > scores).astype(jnp.float32)
        eq = jnp.where((sj == scores) & (lane > j), 1.0, 0.0)
        rank = rank + gt + eq
    mask = (rank < n_active).astype(jnp.float32)

    w = probs * mask
    w = w / (jnp.sum(w, axis=-1, keepdims=True) + 1e-8)
    sign = jnp.where(lane < n_camp_a, 1.0, -1.0)
    coef_ref[...] = w * sign


def _moe_body(coef_ref, x_ref, w1_ref, b1_ref, w2_ref, b2_ref, out_ref):
    e = pl.program_id(0)
    h = pl.program_id(1)

    @pl.when((e == 0) & (h == 0))
    def _init():
        out_ref[...] = jnp.zeros_like(out_ref)

    lane = jax.lax.broadcasted_iota(jnp.int32, coef_ref.shape, 1)
    csel = jnp.sum(
        jnp.where(lane == e, coef_ref[...], 0.0), axis=1, keepdims=True
    )

    hc = jnp.dot(x_ref[...], w1_ref[0], preferred_element_type=jnp.float32)
    hc = jnp.maximum(hc + b1_ref[0], 0.0)
    hcb = (hc * csel).astype(jnp.bfloat16)

    @pl.when(h == 0)
    def _bias2():
        out_ref[...] += csel * b2_ref[0]

    out_ref[...] += jnp.dot(hcb, w2_ref[0], preferred_element_type=jnp.float32)


def kernel(x, gate_w, gate_b, w1, b1, w2, b2):
    n_tok, d_in = x.shape
    e_num, _, d_hid = w1.shape
    d_out = w2.shape[2]
    n_active = max(1, int(e_num * 0.625))
    n_camp_a = e_num // 2

    gwp = jnp.zeros((d_in, _LANES), jnp.float32).at[:, :e_num].set(gate_w)
    gbp = (
        jnp.full((1, _LANES), -1e30, jnp.float32)
        .at[0, :e_num]
        .set(gate_b.astype(jnp.float32))
    )
    coef = pl.pallas_call(
        functools.partial(_routing_body, n_active=n_active, n_camp_a=n_camp_a),
        out_shape=jax.ShapeDtypeStruct((n_tok, _LANES), jnp.float32),
    )(x, gwp, gbp)

    th = min(512, d_hid)
    grid = (e_num, d_hid // th)

    xb = x.astype(jnp.bfloat16)
    w1b = w1.astype(jnp.bfloat16)
    w2b = w2.astype(jnp.bfloat16)
    b1r = b1.reshape(e_num, 1, d_hid).astype(jnp.float32)
    b2r = b2.reshape(e_num, 1, d_out).astype(jnp.float32)

    out = pl.pallas_call(
        _moe_body,
        grid=grid,
        in_specs=[
            pl.BlockSpec((n_tok, _LANES), lambda e, h: (0, 0)),
            pl.BlockSpec((n_tok, d_in), lambda e, h: (0, 0)),
            pl.BlockSpec((1, d_in, th), lambda e, h: (e, 0, h)),
            pl.BlockSpec((1, 1, th), lambda e, h: (e, 0, h)),
            pl.BlockSpec((1, th, d_out), lambda e, h: (e, h, 0)),
            pl.BlockSpec((1, 1, d_out), lambda e, h: (e, 0, 0)),
        ],
        out_specs=pl.BlockSpec((n_tok, d_out), lambda e, h: (0, 0)),
        out_shape=jax.ShapeDtypeStruct((n_tok, d_out), jnp.float32),
        compiler_params=pltpu.CompilerParams(
            dimension_semantics=("arbitrary", "arbitrary")
        ),
    )(coef, xb, w1b, b1r, w2b, b2r)
    return out


# speedup vs baseline: 1.1432x; 1.1432x over previous
import functools
import math

import jax
import jax.numpy as jnp
from jax.experimental import pallas as pl
from jax.experimental.pallas import tpu as pltpu

_TEMP = math.e
_LANES = 128


def _routing_body(x_ref, gw_ref, gb_ref, coef_ref, xb_ref, *, n_active, n_camp_a):
    scores = jnp.dot(x_ref[...], gw_ref[...], preferred_element_type=jnp.float32)
    scores = (scores + gb_ref[...]) * (1.0 / _TEMP)
    m = jnp.max(scores, axis=-1, keepdims=True)
    ex = jnp.exp(scores - m)
    probs = ex / jnp.sum(ex, axis=-1, keepdims=True)

    lane = jax.lax.broadcasted_iota(jnp.int32, scores.shape, 1)
    rank = jnp.zeros(scores.shape, jnp.float32)
    for j in range(8):
        sj = scores[:, j : j + 1]
        gt = (sj > scores).astype(jnp.float32)
        eq = jnp.where((sj == scores) & (lane > j), 1.0, 0.0)
        rank = rank + gt + eq
    mask = (rank < n_active).astype(jnp.float32)

    w = probs * mask
    w = w / (jnp.sum(w, axis=-1, keepdims=True) + 1e-8)
    sign = jnp.where(lane < n_camp_a, 1.0, -1.0)
    coef_ref[...] = w * sign
    xb_ref[...] = x_ref[...].astype(jnp.bfloat16)


def _csel(coef_ref, e):
    lane = jax.lax.broadcasted_iota(jnp.int32, coef_ref.shape, 1)
    return jnp.sum(jnp.where(lane == e, coef_ref[...], 0.0), axis=1, keepdims=True)


def _hidden_body(coef_ref, xb_ref, w1_ref, b1_ref, w2_ref, h_ref, w2b_ref, csel_ref):
    e = pl.program_id(0)
    h = pl.program_id(1)

    @pl.when(h == 0)
    def _():
        csel_ref[...] = _csel(coef_ref, e)

    hc = jnp.dot(xb_ref[...], w1_ref[0].astype(jnp.bfloat16),
                 preferred_element_type=jnp.float32)
    hc = jnp.maximum(hc + b1_ref[0], 0.0)
    h_ref[0] = (hc * csel_ref[...]).astype(jnp.bfloat16)
    w2b_ref[0] = w2_ref[0].astype(jnp.bfloat16)


def _combine_body(coef_ref, h_ref, w2b_ref, b2_ref, out_ref, csel_ref):
    e = pl.program_id(0)
    kh = pl.program_id(1)
    o = pl.program_id(2)

    @pl.when((kh == 0) & (o == 0))
    def _():
        csel_ref[...] = _csel(coef_ref, e)

    t = jnp.dot(h_ref[0], w2b_ref[0], preferred_element_type=jnp.float32)

    @pl.when(kh == 0)
    def _bias():
        t2 = t + csel_ref[...] * b2_ref[0]

        @pl.when(e == 0)
        def _init():
            out_ref[o] = t2

        @pl.when(e > 0)
        def _acc():
            out_ref[o] += t2

    @pl.when(kh > 0)
    def _acc_kh():
        out_ref[o] += t


def kernel(x, gate_w, gate_b, w1, b1, w2, b2):
    n_tok, d_in = x.shape
    e_num, _, d_hid = w1.shape
    d_out = w2.shape[2]
    n_active = max(1, int(e_num * 0.625))
    n_camp_a = e_num // 2

    gwp = jnp.zeros((d_in, _LANES), jnp.float32).at[:, :e_num].set(gate_w)
    gbp = (
        jnp.full((1, _LANES), -1e30, jnp.float32)
        .at[0, :e_num]
        .set(gate_b.astype(jnp.float32))
    )
    coef, xb = pl.pallas_call(
        functools.partial(_routing_body, n_active=n_active, n_camp_a=n_camp_a),
        out_shape=(
            jax.ShapeDtypeStruct((n_tok, _LANES), jnp.float32),
            jax.ShapeDtypeStruct((n_tok, d_in), jnp.bfloat16),
        ),
    )(x, gwp, gbp)

    b1r = b1.reshape(e_num, 1, d_hid).astype(jnp.float32)
    b2r = b2.reshape(e_num, 1, d_out).astype(jnp.float32)

    th = min(512, d_hid)
    h_all, w2b = pl.pallas_call(
        _hidden_body,
        grid=(e_num, d_hid // th),
        in_specs=[
            pl.BlockSpec((n_tok, _LANES), lambda e, h: (0, 0)),
            pl.BlockSpec((n_tok, d_in), lambda e, h: (0, 0)),
            pl.BlockSpec((1, d_in, th), lambda e, h: (e, 0, h)),
            pl.BlockSpec((1, 1, th), lambda e, h: (e, 0, h)),
            pl.BlockSpec((1, th, d_out), lambda e, h: (e, h, 0)),
        ],
        out_specs=(
            pl.BlockSpec((1, n_tok, th), lambda e, h: (e, 0, h)),
            pl.BlockSpec((1, th, d_out), lambda e, h: (e, h, 0)),
        ),
        out_shape=(
            jax.ShapeDtypeStruct((e_num, n_tok, d_hid), jnp.bfloat16),
            jax.ShapeDtypeStruct((e_num, d_hid, d_out), jnp.bfloat16),
        ),
        scratch_shapes=[pltpu.VMEM((n_tok, 1), jnp.float32)],
        compiler_params=pltpu.CompilerParams(
            dimension_semantics=("arbitrary", "arbitrary")
        ),
    )(coef, xb, w1, b1r, w2)

    to = min(256, d_out)
    n_o = d_out // to
    khc = min(2048, d_hid)
    n_kh = d_hid // khc
    out3 = pl.pallas_call(
        _combine_body,
        grid=(e_num, n_kh, n_o),
        in_specs=[
            pl.BlockSpec((n_tok, _LANES), lambda e, kh, o: (0, 0)),
            pl.BlockSpec((1, n_tok, khc), lambda e, kh, o: (e, 0, kh)),
            pl.BlockSpec((1, khc, to), lambda e, kh, o: (e, kh, o)),
            pl.BlockSpec((1, 1, to), lambda e, kh, o: (e, 0, o)),
        ],
        out_specs=pl.BlockSpec((n_o, n_tok, to), lambda e, kh, o: (0, 0, 0)),
        out_shape=jax.ShapeDtypeStruct((n_o, n_tok, to), jnp.float32),
        scratch_shapes=[pltpu.VMEM((n_tok, 1), jnp.float32)],
        compiler_params=pltpu.CompilerParams(
            dimension_semantics=("arbitrary", "arbitrary", "arbitrary")
        ),
    )(coef, h_all, w2b, b2r)
    return out3.transpose(1, 0, 2).reshape(n_tok, d_out)
